# baseline (speedup 1.0000x reference)
import functools

import jax
import jax.numpy as jnp
from jax import lax
from jax.experimental import pallas as pl
from jax.experimental.pallas import tpu as pltpu
from jax.experimental.pallas import tpu_sc as plsc

_M = 1000000
_B = 16384
_F = 26
_K = 32
_NC = 2
_NS = 16
_L = 16
_NW = _NC * _NS
_EW = _B // _NW
_C = 16
_NCH = _EW // _C
_NR = _C * _F


def _fm_body(w_hbm, vs_hbm, b_hbm, out_hbm, idx_v, idx4_v, rows_v, brow_v,
             out_v, sem_r, sem_b):
    cid = lax.axis_index("c")
    sid = lax.axis_index("s")
    wid = sid * _NC + cid
    iota = lax.iota(jnp.int32, _L)
    row_step = iota * _F

    def chunk_body(c, carry):
        ebase = wid * _EW + c * _C
        pltpu.sync_copy(w_hbm.at[pl.ds(ebase * _F, _NR)], idx_v)
        for j in range(_NR // _L):
            idx4_v[pl.ds(j * _L, _L)] = lax.shift_right_logical(
                idx_v[pl.ds(j * _L, _L)], 2
            )
        cp_r = pltpu.async_copy(vs_hbm.at[idx4_v], rows_v, sem_r)
        cp_b = pltpu.async_copy(b_hbm.at[idx_v], brow_v, sem_b)
        cp_r.wait()
        cp_b.wait()
        for g in range(_C // _L):
            row0 = row_step + (g * _L * _F)
            lin = plsc.load_gather(brow_v, [row0])
            for f in range(1, _F):
                lin = lin + plsc.load_gather(brow_v, [row0 + f])

            def k_body(k, kc):
                pw, q = kc
                kcol = jnp.broadcast_to(k, (_L,))
                sk = None
                for f in range(_F):
                    iv = plsc.load_gather(idx_v, [row0 + f])
                    col = lax.shift_left(
                        lax.bitwise_and(iv, jnp.int32(3)), 5
                    ) + kcol
                    v = plsc.load_gather(rows_v, [row0 + f, col])
                    sk = v if sk is None else sk + v
                    q = q + v * v
                pw = pw + sk * sk
                return pw, q

            zero = jnp.zeros((_L,), jnp.float32)
            pw, q = lax.fori_loop(0, _K, k_body, (zero, zero))
            out_v[pl.ds(g * _L, _L)] = 0.5 * (pw - q) + lin
        pltpu.sync_copy(out_v, out_hbm.at[pl.ds(ebase, _C)])
        return carry

    lax.fori_loop(0, _NCH, chunk_body, 0)


_fm = functools.partial(
    pl.kernel,
    mesh=plsc.VectorSubcoreMesh(core_axis_name="c", subcore_axis_name="s"),
    out_type=jax.ShapeDtypeStruct((_B,), jnp.float32),
    compiler_params=pltpu.CompilerParams(
        needs_layout_passes=False, use_tc_tiling_on_sc=False
    ),
    scratch_types=[
        pltpu.VMEM((_NR,), jnp.int32),
        pltpu.VMEM((_NR,), jnp.int32),
        pltpu.VMEM((_NR, 4 * _K), jnp.float32),
        pltpu.VMEM((_NR,), jnp.float32),
        pltpu.VMEM((_C,), jnp.float32),
        pltpu.SemaphoreType.DMA,
        pltpu.SemaphoreType.DMA,
    ],
)(_fm_body)


@jax.jit
def kernel(w_nz, vs, biases, bias):
    w_flat = w_nz.reshape(-1).astype(jnp.int32)
    vs4 = vs.reshape(_M // 4, 4 * _K)
    out = _fm(w_flat, vs4, biases)
    return out + bias[0]


# speedup vs baseline: 1.0290x; 1.0290x over previous
import functools

import jax
import jax.numpy as jnp
from jax import lax
from jax.experimental import pallas as pl
from jax.experimental.pallas import tpu as pltpu
from jax.experimental.pallas import tpu_sc as plsc

_M = 1000000
_B = 16384
_F = 26
_K = 32
_NC = 2
_NS = 16
_L = 16
_NW = _NC * _NS
_EW = _B // _NW
_C = 16
_NCH = _EW // _C
_NR = _C * _F


def _fm_body(w_hbm, vs_hbm, b_hbm, out_hbm, idx_v, idx4_v, rows_v, brow_v,
             out_v, sem_r, sem_b):
    cid = lax.axis_index("c")
    sid = lax.axis_index("s")
    wid = sid * _NC + cid
    iota = lax.iota(jnp.int32, _L)
    row_step = iota * _F

    def chunk_body(c, carry):
        ebase = wid * _EW + c * _C
        pltpu.sync_copy(w_hbm.at[pl.ds(ebase * _F, _NR)], idx_v)
        for j in range(_NR // _L):
            idx4_v[pl.ds(j * _L, _L)] = lax.shift_right_logical(
                idx_v[pl.ds(j * _L, _L)], 2
            )
        cp_r = pltpu.async_copy(vs_hbm.at[idx4_v], rows_v, sem_r)
        cp_b = pltpu.async_copy(b_hbm.at[idx_v], brow_v, sem_b)
        cp_r.wait()
        cp_b.wait()
        for g in range(_C // _L):
            row0 = row_step + (g * _L * _F)
            lin = plsc.load_gather(brow_v, [row0])
            for f in range(1, _F):
                lin = lin + plsc.load_gather(brow_v, [row0 + f])

            def k_body(k, kc):
                pw, q = kc
                kcol = jnp.broadcast_to(k, (_L,))
                sk = None
                for f in range(_F):
                    iv = plsc.load_gather(idx_v, [row0 + f])
                    col = lax.shift_left(
                        lax.bitwise_and(iv, jnp.int32(3)), 5
                    ) + kcol
                    v = plsc.load_gather(rows_v, [row0 + f, col])
                    sk = v if sk is None else sk + v
                    q = q + v * v
                pw = pw + sk * sk
                return pw, q

            zero = jnp.zeros((_L,), jnp.float32)
            pw, q = lax.fori_loop(0, _K, k_body, (zero, zero))
            out_v[pl.ds(g * _L, _L)] = 0.5 * (pw - q) + lin
        pltpu.sync_copy(out_v, out_hbm.at[pl.ds(ebase, _C)])
        return carry

    lax.fori_loop(0, _NCH, chunk_body, 0)


_fm = functools.partial(
    pl.kernel,
    mesh=plsc.VectorSubcoreMesh(core_axis_name="c", subcore_axis_name="s"),
    out_type=jax.ShapeDtypeStruct((_B,), jnp.float32),
    compiler_params=pltpu.CompilerParams(
        needs_layout_passes=False, use_tc_tiling_on_sc=True
    ),
    scratch_types=[
        pltpu.VMEM((_NR,), jnp.int32),
        pltpu.VMEM((_NR,), jnp.int32),
        pltpu.VMEM((_NR, 4 * _K), jnp.float32),
        pltpu.VMEM((_NR,), jnp.float32),
        pltpu.VMEM((_C,), jnp.float32),
        pltpu.SemaphoreType.DMA,
        pltpu.SemaphoreType.DMA,
    ],
)(_fm_body)


@jax.jit
def kernel(w_nz, vs, biases, bias):
    w_flat = w_nz.reshape(-1).astype(jnp.int32)
    vs4 = vs.reshape(_M // 4, 4 * _K)
    out = _fm(w_flat, vs4, biases)
    return out + bias[0]


# speedup vs baseline: 1.1548x; 1.1223x over previous
import functools

import jax
import jax.numpy as jnp
from jax import lax
from jax.experimental import pallas as pl
from jax.experimental.pallas import tpu as pltpu
from jax.experimental.pallas import tpu_sc as plsc

_M = 1000000
_B = 16384
_F = 26
_K = 32
_NC = 2
_NS = 16
_L = 16
_NW = _NC * _NS
_EW = _B // _NW
_C = 32
_NCH = _EW // _C
_NR = _C * _F


def _fm_body(w_hbm, vs_hbm, b_hbm, out_hbm, idx_v, rows_v, brow_v, out_v,
             sem_r, sem_b):
    cid = lax.axis_index("c")
    sid = lax.axis_index("s")
    wid = sid * _NC + cid
    iota = lax.iota(jnp.int32, _L)
    row_step = iota * _F

    def chunk_body(c, carry):
        ebase = wid * _EW + c * _C
        pltpu.sync_copy(w_hbm.at[pl.ds(ebase * _F, _NR)], idx_v)
        cp_r = pltpu.async_copy(vs_hbm.at[idx_v], rows_v, sem_r)
        cp_b = pltpu.async_copy(b_hbm.at[idx_v], brow_v, sem_b)
        cp_r.wait()
        cp_b.wait()
        for g in range(_C // _L):
            row0 = row_step + (g * _L * _F)
            lin = plsc.load_gather(brow_v, [row0])
            for f in range(1, _F):
                lin = lin + plsc.load_gather(brow_v, [row0 + f])

            def k_body(k, kc):
                pw, q = kc
                kcol = jnp.broadcast_to(k, (_L,))
                sk = plsc.load_gather(rows_v, [row0, kcol])
                q = q + sk * sk
                for f in range(1, _F):
                    v = plsc.load_gather(rows_v, [row0 + f, kcol])
                    sk = sk + v
                    q = q + v * v
                pw = pw + sk * sk
                return pw, q

            zero = jnp.zeros((_L,), jnp.float32)
            pw, q = lax.fori_loop(0, _K, k_body, (zero, zero))
            out_v[pl.ds(g * _L, _L)] = 0.5 * (pw - q) + lin
        pltpu.sync_copy(out_v, out_hbm.at[pl.ds(ebase, _C)])
        return carry

    lax.fori_loop(0, _NCH, chunk_body, 0)


_fm = functools.partial(
    pl.kernel,
    mesh=plsc.VectorSubcoreMesh(core_axis_name="c", subcore_axis_name="s"),
    out_type=jax.ShapeDtypeStruct((_B,), jnp.float32),
    compiler_params=pltpu.CompilerParams(
        needs_layout_passes=False, use_tc_tiling_on_sc=False
    ),
    scratch_types=[
        pltpu.VMEM((_NR,), jnp.int32),
        pltpu.VMEM((_NR, _K), jnp.float32),
        pltpu.VMEM((_NR,), jnp.float32),
        pltpu.VMEM((_C,), jnp.float32),
        pltpu.SemaphoreType.DMA,
        pltpu.SemaphoreType.DMA,
    ],
)(_fm_body)


@jax.jit
def kernel(w_nz, vs, biases, bias):
    w_flat = w_nz.reshape(-1).astype(jnp.int32)
    vs_lin = (vs.reshape(-1) * jnp.float32(1.0)).reshape(_M, _K)
    out = _fm(w_flat, vs_lin, biases)
    return out + bias[0]
